# Initial kernel scaffold; baseline (speedup 1.0000x reference)
import jax
import jax.numpy as jnp
from jax.experimental import pallas as pl


def kernel(x, W_pre, W_spec, P):
    raise NotImplementedError("write your pallas kernel here")


# speedup vs baseline: 3.6715x; 3.6715x over previous
import functools

import jax
import jax.numpy as jnp
from jax import lax
from jax.experimental import pallas as pl
from jax.experimental.pallas import tpu as pltpu
from jax.experimental.pallas import tpu_sc as plsc

VOCAB = 100000
EMBED = 64
WINDOW = 200
NUM_SPEC = 10
BATCH = 1024

NC = 2
NS = 16
NW = NC * NS
ROWS_PER_W = BATCH // NW

WPAD = 208
NVEC = WPAD // 16


def _body(x_hbm, wpre_hbm, wspec_hbm, p_hbm, out_hbm,
          xv, row_buf, p_vmem, spec_vmem, sem):
  wid = lax.axis_index("s") * NC + lax.axis_index("c")
  base = wid * ROWS_PER_W

  pltpu.sync_copy(p_hbm, p_vmem)
  pltpu.sync_copy(wspec_hbm, spec_vmem)

  def row_step(r, carry):
    row = base + r
    pltpu.sync_copy(x_hbm.at[pl.ds(row * WINDOW, WINDOW)], xv.at[pl.ds(0, WINDOW)])

    c1 = pltpu.async_copy(wpre_hbm.at[xv.at[pl.ds(0, 128)]],
                          row_buf.at[pl.ds(0, 128)], sem)
    c2 = pltpu.async_copy(wpre_hbm.at[xv.at[pl.ds(128, WINDOW - 128)]],
                          row_buf.at[pl.ds(128, WINDOW - 128)], sem)
    c1.wait()
    c2.wait()

    def add_step(t, carry2):
      for k in range(EMBED // 16):
        sl = pl.ds(16 * k, 16)
        row_buf[t, sl] = row_buf[t, sl] + p_vmem[t, sl]
      return carry2

    lax.fori_loop(0, WINDOW, add_step, 0, unroll=4)

    lanes = lax.iota(jnp.int32, 16)
    acc = jnp.zeros((16,), dtype=jnp.int32)
    for g in range(NVEC):
      ids = xv[pl.ds(16 * g, 16)]
      m = ids < NUM_SPEC
      if 16 * (g + 1) > WINDOW:
        m = jnp.logical_and(m, lanes < (WINDOW - 16 * g))
      acc = acc | jnp.where(m, 1, 0).astype(jnp.int32)
    nspec = acc[0]
    for i in range(1, 16):
      nspec = nspec + acc[i]

    @pl.when(nspec > 0)
    def _patch():
      def patch_step(g, carry2):
        ids = xv[pl.ds(16 * g, 16)]
        for i in range(16):
          s = ids[i]
          t = 16 * g + i

          @pl.when(jnp.logical_and(s < NUM_SPEC, t < WINDOW))
          def _fix():
            for k in range(EMBED // 16):
              sl = pl.ds(16 * k, 16)
              row_buf[t, sl] = spec_vmem[s, sl] + p_vmem[t, sl]

        return carry2

      lax.fori_loop(0, NVEC, patch_step, 0)

    pltpu.sync_copy(row_buf, out_hbm.at[pl.ds(row * WINDOW, WINDOW)])
    return carry

  lax.fori_loop(0, ROWS_PER_W, row_step, 0)


@jax.jit
def _run(x, W_pre, W_spec, P):
  mesh = plsc.VectorSubcoreMesh(core_axis_name="c", subcore_axis_name="s")
  f = pl.kernel(
      _body,
      out_type=jax.ShapeDtypeStruct((BATCH * WINDOW, EMBED), jnp.float32),
      mesh=mesh,
      scratch_types=[
          pltpu.VMEM((WPAD,), jnp.int32),
          pltpu.VMEM((WINDOW, EMBED), jnp.float32),
          pltpu.VMEM((WINDOW, EMBED), jnp.float32),
          pltpu.VMEM((NUM_SPEC, EMBED), jnp.float32),
          pltpu.SemaphoreType.DMA,
      ],
      compiler_params=pltpu.CompilerParams(use_tc_tiling_on_sc=False),
  )
  out = f(x.reshape(-1), W_pre, W_spec, P)
  return out.reshape(BATCH, WINDOW, EMBED)


def kernel(x, W_pre, W_spec, P):
  return _run(x.astype(jnp.int32), W_pre, W_spec, P)


# speedup vs baseline: 4.1846x; 1.1398x over previous
import functools

import jax
import jax.numpy as jnp
from jax import lax
from jax.experimental import pallas as pl
from jax.experimental.pallas import tpu as pltpu
from jax.experimental.pallas import tpu_sc as plsc

VOCAB = 100000
EMBED = 64
WINDOW = 200
NUM_SPEC = 10
BATCH = 1024

NC = 2
NS = 16
NW = NC * NS
ROWS_PER_W = BATCH // NW
SLAB = ROWS_PER_W * WINDOW

NVEC = (WINDOW + 15) // 16


def _body(x_hbm, wpre_hbm, wspec_hbm, p_hbm, out_hbm,
          xs, row_buf0, row_buf1, p_vmem, spec_vmem, sem0, sem1):
  wid = lax.axis_index("s") * NC + lax.axis_index("c")
  base = wid * ROWS_PER_W

  pltpu.sync_copy(x_hbm.at[pl.ds(base * WINDOW, SLAB)], xs)
  pltpu.sync_copy(p_hbm, p_vmem)
  pltpu.sync_copy(wspec_hbm, spec_vmem)

  bufs = (row_buf0, row_buf1)
  sems = (sem0, sem1)

  def gather(slot, r):
    off = r * WINDOW
    pltpu.make_async_copy(wpre_hbm.at[xs.at[pl.ds(off, 128)]],
                          bufs[slot].at[pl.ds(0, 128)], sems[slot]).start()
    pltpu.make_async_copy(
        wpre_hbm.at[xs.at[pl.ds(off + 128, WINDOW - 128)]],
        bufs[slot].at[pl.ds(128, WINDOW - 128)], sems[slot]).start()

  def wait(slot, r):
    off = r * WINDOW
    pltpu.make_async_copy(wpre_hbm.at[xs.at[pl.ds(off, 128)]],
                          bufs[slot].at[pl.ds(0, 128)], sems[slot]).wait()
    pltpu.make_async_copy(
        wpre_hbm.at[xs.at[pl.ds(off + 128, WINDOW - 128)]],
        bufs[slot].at[pl.ds(128, WINDOW - 128)], sems[slot]).wait()

  def process(slot, r):
    row_buf = bufs[slot]

    def add_step(t, carry):
      for k in range(EMBED // 16):
        sl = pl.ds(16 * k, 16)
        row_buf[t, sl] = row_buf[t, sl] + p_vmem[t, sl]
      return carry

    lax.fori_loop(0, WINDOW, add_step, 0, unroll=4)

    lanes = lax.iota(jnp.int32, 16)
    acc = jnp.zeros((16,), dtype=jnp.int32)
    for g in range(NVEC):
      n_here = min(16, WINDOW - 16 * g)
      ids = xs[pl.ds(r * WINDOW + 16 * g, 16)]
      m = ids < NUM_SPEC
      if n_here < 16:
        m = jnp.logical_and(m, lanes < n_here)
      acc = acc | jnp.where(m, 1, 0).astype(jnp.int32)
    nspec = acc[0]
    for i in range(1, 16):
      nspec = nspec + acc[i]

    @pl.when(nspec > 0)
    def _patch():
      def patch_step(g, carry):
        ids = xs[pl.ds(r * WINDOW + 16 * g, 16)]
        for i in range(16):
          s = ids[i]
          t = 16 * g + i

          @pl.when(jnp.logical_and(s < NUM_SPEC, t < WINDOW))
          def _fix():
            for k in range(EMBED // 16):
              sl = pl.ds(16 * k, 16)
              row_buf[t, sl] = spec_vmem[s, sl] + p_vmem[t, sl]

        return carry

      lax.fori_loop(0, NVEC, patch_step, 0)

    pltpu.sync_copy(row_buf, out_hbm.at[pl.ds((base + r) * WINDOW, WINDOW)])

  gather(0, 0)

  def pair_step(i, carry):
    ra = 2 * i
    rb = 2 * i + 1
    gather(1, rb)
    wait(0, ra)
    process(0, ra)

    @pl.when(i < ROWS_PER_W // 2 - 1)
    def _prefetch():
      gather(0, ra + 2)

    wait(1, rb)
    process(1, rb)
    return carry

  lax.fori_loop(0, ROWS_PER_W // 2, pair_step, 0)


@jax.jit
def _run(x, W_pre, W_spec, P):
  mesh = plsc.VectorSubcoreMesh(core_axis_name="c", subcore_axis_name="s")
  f = pl.kernel(
      _body,
      out_type=jax.ShapeDtypeStruct((BATCH * WINDOW, EMBED), jnp.float32),
      mesh=mesh,
      scratch_types=[
          pltpu.VMEM((SLAB,), jnp.int32),
          pltpu.VMEM((WINDOW, EMBED), jnp.float32),
          pltpu.VMEM((WINDOW, EMBED), jnp.float32),
          pltpu.VMEM((WINDOW, EMBED), jnp.float32),
          pltpu.VMEM((NUM_SPEC, EMBED), jnp.float32),
          pltpu.SemaphoreType.DMA,
          pltpu.SemaphoreType.DMA,
      ],
      compiler_params=pltpu.CompilerParams(use_tc_tiling_on_sc=False),
  )
  out = f(x.reshape(-1), W_pre, W_spec, P)
  return out.reshape(BATCH, WINDOW, EMBED)


def kernel(x, W_pre, W_spec, P):
  return _run(x.astype(jnp.int32), W_pre, W_spec, P)


# speedup vs baseline: 5.2569x; 1.2563x over previous
import functools

import jax
import jax.numpy as jnp
from jax import lax
from jax.experimental import pallas as pl
from jax.experimental.pallas import tpu as pltpu
from jax.experimental.pallas import tpu_sc as plsc

VOCAB = 100000
EMBED = 64
WINDOW = 200
NUM_SPEC = 10
BATCH = 1024

NC = 2
NS = 16
NW = NC * NS
ROWS_PER_W = BATCH // NW
RPB = 4
BLK = RPB * WINDOW
NBLK = ROWS_PER_W // RPB
NGRP = BLK // 16
CHUNKS = [(0, 128), (128, 128), (256, 128), (384, 128),
          (512, 128), (640, 128), (768, 32)]


def _body(x_hbm, wpre_hbm, wspec_hbm, p_hbm, out_hbm,
          xs, buf0, buf1, p_vmem, spec_vmem, gsem0, gsem1, osem0, osem1):
  wid = lax.axis_index("s") * NC + lax.axis_index("c")
  base = wid * ROWS_PER_W

  pltpu.sync_copy(x_hbm.at[pl.ds(base * WINDOW, ROWS_PER_W * WINDOW)], xs)
  pltpu.sync_copy(p_hbm, p_vmem)
  pltpu.sync_copy(wspec_hbm, spec_vmem)

  bufs = (buf0, buf1)
  gsems = (gsem0, gsem1)
  osems = (osem0, osem1)

  def gather_copies(slot, b):
    off = b * BLK
    return [
        pltpu.make_async_copy(wpre_hbm.at[xs.at[pl.ds(off + o, n)]],
                              bufs[slot].at[pl.ds(o, n)], gsems[slot])
        for o, n in CHUNKS
    ]

  def start_gather(slot, b):
    for c in gather_copies(slot, b):
      c.start()

  def wait_gather(slot, b):
    for c in gather_copies(slot, b):
      c.wait()

  def out_copy(slot, b):
    return pltpu.make_async_copy(
        bufs[slot], out_hbm.at[pl.ds((base + RPB * b) * WINDOW, BLK)],
        osems[slot])

  def process(slot, b):
    buf = bufs[slot]

    def add_step(t, carry):
      for k in range(EMBED // 16):
        sl = pl.ds(16 * k, 16)
        pv = p_vmem[t, sl]
        for rr in range(RPB):
          buf[rr * WINDOW + t, sl] = buf[rr * WINDOW + t, sl] + pv
      return carry

    lax.fori_loop(0, WINDOW, add_step, 0, unroll=4)

    acc = jnp.zeros((16,), dtype=jnp.int32)
    for g in range(NGRP):
      ids = xs[pl.ds(b * BLK + 16 * g, 16)]
      acc = acc | jnp.where(ids < NUM_SPEC, 1, 0).astype(jnp.int32)
    nspec = acc[0]
    for i in range(1, 16):
      nspec = nspec + acc[i]

    @pl.when(nspec > 0)
    def _patch():
      def patch_step(g, carry):
        ids = xs[pl.ds(b * BLK + 16 * g, 16)]
        for i in range(16):
          s = ids[i]
          t = 16 * g + i
          pos = lax.rem(t, WINDOW)

          @pl.when(s < NUM_SPEC)
          def _fix():
            for k in range(EMBED // 16):
              sl = pl.ds(16 * k, 16)
              buf[t, sl] = spec_vmem[s, sl] + p_vmem[pos, sl]

        return carry

      lax.fori_loop(0, NGRP, patch_step, 0)

  start_gather(0, 0)

  def pair_step(i, carry):
    ba = 2 * i
    bb = 2 * i + 1

    @pl.when(i > 0)
    def _drain1():
      out_copy(1, bb - 2).wait()

    start_gather(1, bb)

    wait_gather(0, ba)
    process(0, ba)
    out_copy(0, ba).start()

    wait_gather(1, bb)
    process(1, bb)
    out_copy(1, bb).start()

    @pl.when(i < NBLK // 2 - 1)
    def _next0():
      out_copy(0, ba).wait()
      start_gather(0, ba + 2)

    return carry

  lax.fori_loop(0, NBLK // 2, pair_step, 0)

  out_copy(0, NBLK - 2).wait()
  out_copy(1, NBLK - 1).wait()


@jax.jit
def _run(x, W_pre, W_spec, P):
  mesh = plsc.VectorSubcoreMesh(core_axis_name="c", subcore_axis_name="s")
  f = pl.kernel(
      _body,
      out_type=jax.ShapeDtypeStruct((BATCH * WINDOW, EMBED), jnp.float32),
      mesh=mesh,
      scratch_types=[
          pltpu.VMEM((ROWS_PER_W * WINDOW,), jnp.int32),
          pltpu.VMEM((BLK, EMBED), jnp.float32),
          pltpu.VMEM((BLK, EMBED), jnp.float32),
          pltpu.VMEM((WINDOW, EMBED), jnp.float32),
          pltpu.VMEM((NUM_SPEC, EMBED), jnp.float32),
          pltpu.SemaphoreType.DMA,
          pltpu.SemaphoreType.DMA,
          pltpu.SemaphoreType.DMA,
          pltpu.SemaphoreType.DMA,
      ],
      compiler_params=pltpu.CompilerParams(use_tc_tiling_on_sc=False),
  )
  out = f(x.reshape(-1), W_pre, W_spec, P)
  return out.reshape(BATCH, WINDOW, EMBED)


def kernel(x, W_pre, W_spec, P):
  return _run(x.astype(jnp.int32), W_pre, W_spec, P)


# speedup vs baseline: 5.2588x; 1.0004x over previous
import functools

import jax
import jax.numpy as jnp
from jax import lax
from jax.experimental import pallas as pl
from jax.experimental.pallas import tpu as pltpu
from jax.experimental.pallas import tpu_sc as plsc

VOCAB = 100000
EMBED = 64
WINDOW = 200
NUM_SPEC = 10
BATCH = 1024

NC = 2
NS = 16
NW = NC * NS
ROWS_PER_W = BATCH // NW
RPB = 4
NBLK = ROWS_PER_W // RPB
GRP_OFFS = [16 * g for g in range(WINDOW // 16)] + [WINDOW - 16]
NVEC = len(GRP_OFFS)
ROW_CHUNKS = [(0, 128), (128, WINDOW - 128)]


def _body(x_hbm, wpre_hbm, wspec_hbm, p_hbm, out_hbm,
          xs, buf0, buf1, p_vmem, spec_vmem, gsem0, gsem1, osem0, osem1):
  wid = lax.axis_index("s") * NC + lax.axis_index("c")
  base = wid * ROWS_PER_W

  pltpu.sync_copy(x_hbm.at[pl.ds(base, ROWS_PER_W)], xs)
  pltpu.sync_copy(p_hbm, p_vmem)
  pltpu.sync_copy(wspec_hbm, spec_vmem)

  bufs = (buf0, buf1)
  gsems = (gsem0, gsem1)
  osems = (osem0, osem1)

  def gather_copies(slot, b):
    cs = []
    for rr in range(RPB):
      r = RPB * b + rr
      for o, n in ROW_CHUNKS:
        cs.append(pltpu.make_async_copy(
            wpre_hbm.at[xs.at[r, pl.ds(o, n)]],
            bufs[slot].at[rr].at[pl.ds(o, n)], gsems[slot]))
    return cs

  def start_gather(slot, b):
    for c in gather_copies(slot, b):
      c.start()

  def wait_gather(slot, b):
    for c in gather_copies(slot, b):
      c.wait()

  def out_copy(slot, b):
    return pltpu.make_async_copy(
        bufs[slot], out_hbm.at[pl.ds(base + RPB * b, RPB)], osems[slot])

  def process(slot, b):
    buf = bufs[slot]

    def add_step(t, carry):
      for k in range(EMBED // 16):
        sl = pl.ds(16 * k, 16)
        pv = p_vmem[t, sl]
        for rr in range(RPB):
          buf[rr, t, sl] = buf[rr, t, sl] + pv
      return carry

    lax.fori_loop(0, WINDOW, add_step, 0, unroll=4)

    acc = jnp.zeros((16,), dtype=jnp.int32)
    for rr in range(RPB):
      for off in GRP_OFFS:
        ids = xs[RPB * b + rr, pl.ds(off, 16)]
        acc = acc | jnp.where(ids < NUM_SPEC, 1, 0).astype(jnp.int32)
    nspec = acc[0]
    for i in range(1, 16):
      nspec = nspec + acc[i]

    @pl.when(nspec > 0)
    def _patch():
      def patch_step(g, carry):
        off = pl.multiple_of(jnp.where(g == NVEC - 1, WINDOW - 16, 16 * g), 8)
        for rr in range(RPB):
          ids = xs[RPB * b + rr, pl.ds(off, 16)]
          for i in range(16):
            s = ids[i]
            t = off + i

            @pl.when(s < NUM_SPEC)
            def _fix():
              for k in range(EMBED // 16):
                sl = pl.ds(16 * k, 16)
                buf[rr, t, sl] = spec_vmem[s, sl] + p_vmem[t, sl]

        return carry

      lax.fori_loop(0, NVEC, patch_step, 0)

  start_gather(0, 0)

  def pair_step(i, carry):
    ba = 2 * i
    bb = 2 * i + 1

    @pl.when(i > 0)
    def _drain1():
      out_copy(1, bb - 2).wait()

    start_gather(1, bb)

    wait_gather(0, ba)
    process(0, ba)
    out_copy(0, ba).start()

    wait_gather(1, bb)
    process(1, bb)
    out_copy(1, bb).start()

    @pl.when(i < NBLK // 2 - 1)
    def _next0():
      out_copy(0, ba).wait()
      start_gather(0, ba + 2)

    return carry

  lax.fori_loop(0, NBLK // 2, pair_step, 0)

  out_copy(0, NBLK - 2).wait()
  out_copy(1, NBLK - 1).wait()


@jax.jit
def _run(x, W_pre, W_spec, P):
  mesh = plsc.VectorSubcoreMesh(core_axis_name="c", subcore_axis_name="s")
  f = pl.kernel(
      _body,
      out_type=jax.ShapeDtypeStruct((BATCH, WINDOW, EMBED), jnp.float32),
      mesh=mesh,
      scratch_types=[
          pltpu.VMEM((ROWS_PER_W, WINDOW), jnp.int32),
          pltpu.VMEM((RPB, WINDOW, EMBED), jnp.float32),
          pltpu.VMEM((RPB, WINDOW, EMBED), jnp.float32),
          pltpu.VMEM((WINDOW, EMBED), jnp.float32),
          pltpu.VMEM((NUM_SPEC, EMBED), jnp.float32),
          pltpu.SemaphoreType.DMA,
          pltpu.SemaphoreType.DMA,
          pltpu.SemaphoreType.DMA,
          pltpu.SemaphoreType.DMA,
      ],
      compiler_params=pltpu.CompilerParams(use_tc_tiling_on_sc=False),
  )
  return f(x, W_pre, W_spec, P)


def kernel(x, W_pre, W_spec, P):
  return _run(x.astype(jnp.int32), W_pre, W_spec, P)
